# baseline (speedup 1.0000x reference)
import functools

import jax
import jax.numpy as jnp
from jax import lax
from jax.experimental import pallas as pl
from jax.experimental.pallas import tpu as pltpu
from jax.experimental.pallas import tpu_sc as plsc

D_EMB = 1024
D_MODEL = 2048
N_TOK = 4 * 4096

NC = 2
NS = 16
NW = NC * NS
TW = N_TOK // NW
C = 16
NCHUNK = TW // C
NBUF = 2
NG = NCHUNK // NBUF


def _proj_body(emb_ref, w_ref, b_ref, out_ref):
    out_ref[...] = (
        jnp.dot(emb_ref[...], w_ref[...], preferred_element_type=jnp.float32)
        + b_ref[...][None, :]
    )


def _compute_proj(emb_table, W, b):
    return pl.pallas_call(
        _proj_body,
        out_shape=jax.ShapeDtypeStruct((2, D_MODEL), jnp.float32),
    )(emb_table, W, b)


@functools.partial(
    pl.kernel,
    out_type=jax.ShapeDtypeStruct((N_TOK * D_MODEL,), jnp.float32),
    mesh=plsc.VectorSubcoreMesh(core_axis_name="c", subcore_axis_name="s"),
    compiler_params=pltpu.CompilerParams(needs_layout_passes=False),
    scratch_types=[
        pltpu.VMEM((TW,), jnp.int32),
        pltpu.VMEM((2 * D_MODEL,), jnp.float32),
        pltpu.VMEM((C * D_MODEL,), jnp.float32),
        pltpu.VMEM((C * D_MODEL,), jnp.float32),
        pltpu.SemaphoreType.DMA,
        pltpu.SemaphoreType.DMA,
    ],
)
def _sc_gather(x_hbm, proj_hbm, out_hbm, idx_v, proj_v,
               b0, b1, ws0, ws1):
    cid = lax.axis_index("c")
    sid = lax.axis_index("s")
    wid = sid * NC + cid
    base = wid * TW
    pltpu.sync_copy(x_hbm.at[pl.ds(base, TW)], idx_v)
    pltpu.sync_copy(proj_hbm, proj_v)

    bufs = (b0, b1)
    wsems = (ws0, ws1)
    lanes = lax.iota(jnp.int32, 16)

    def build(ci, p):
        rows = idx_v[pl.ds(ci * C, 16)]
        gbase = rows * D_MODEL
        sbase = lanes * D_MODEL

        def col(c, carry):
            vals = plsc.load_gather(proj_v, [gbase + c])
            plsc.store_scatter(bufs[p], [sbase + c], vals)
            return carry

        lax.fori_loop(0, D_MODEL, col, 0, unroll=16)

    def start_write(ci, p):
        pltpu.async_copy(
            bufs[p], out_hbm.at[pl.ds((base + ci * C) * D_MODEL, C * D_MODEL)],
            wsems[p])

    def wait_write(ci, p):
        pltpu.make_async_copy(
            bufs[p], out_hbm.at[pl.ds((base + ci * C) * D_MODEL, C * D_MODEL)],
            wsems[p]
        ).wait()

    build(0, 0)

    def body(g, carry):
        ci0 = 2 * g
        start_write(ci0, 0)

        @pl.when(g > 0)
        def _():
            wait_write(ci0 - 1, 1)

        build(ci0 + 1, 1)
        start_write(ci0 + 1, 1)
        wait_write(ci0, 0)

        @pl.when(g < NG - 1)
        def _():
            build(ci0 + 2, 0)

        return carry

    lax.fori_loop(0, NG, body, 0)
    wait_write(NCHUNK - 1, 1)


def kernel(x, emb_table, W, b):
    proj = _compute_proj(emb_table, W, b)
    xf = x.reshape(-1).astype(jnp.int32)
    out = _sc_gather(xf, proj.reshape(-1))
    return out.reshape(x.shape[0], x.shape[1], D_MODEL)


# speedup vs baseline: 5.8985x; 5.8985x over previous
import functools

import jax
import jax.numpy as jnp
from jax import lax
from jax.experimental import pallas as pl
from jax.experimental.pallas import tpu as pltpu
from jax.experimental.pallas import tpu_sc as plsc

D_EMB = 1024
D_MODEL = 2048
N_TOK = 4 * 4096

NC = 2
NS = 16
NW = NC * NS
TW = N_TOK // NW
C = 16
NCHUNK = TW // C
NBUF = 2
NG = NCHUNK // NBUF


def _proj_body(emb_ref, w_ref, b_ref, out_ref):
    out_ref[...] = (
        jnp.dot(emb_ref[...], w_ref[...], preferred_element_type=jnp.float32)
        + b_ref[...][None, :]
    )


def _compute_proj(emb_table, W, b):
    return pl.pallas_call(
        _proj_body,
        out_shape=jax.ShapeDtypeStruct((2, D_MODEL), jnp.float32),
    )(emb_table, W, b)


@functools.partial(
    pl.kernel,
    out_type=jax.ShapeDtypeStruct((N_TOK * D_MODEL,), jnp.float32),
    mesh=plsc.VectorSubcoreMesh(core_axis_name="c", subcore_axis_name="s"),
    compiler_params=pltpu.CompilerParams(needs_layout_passes=False),
    scratch_types=[
        pltpu.VMEM((TW,), jnp.int32),
        pltpu.VMEM((2 * D_MODEL,), jnp.float32),
        pltpu.VMEM((C * D_MODEL,), jnp.float32),
        pltpu.VMEM((C * D_MODEL,), jnp.float32),
        pltpu.SemaphoreType.DMA,
        pltpu.SemaphoreType.DMA,
    ],
)
def _sc_gather(x_hbm, proj_hbm, out_hbm, idx_v, proj_v,
               b0, b1, ws0, ws1):
    cid = lax.axis_index("c")
    sid = lax.axis_index("s")
    wid = sid * NC + cid
    base = wid * TW
    pltpu.sync_copy(x_hbm.at[pl.ds(base, TW)], idx_v)
    pltpu.sync_copy(proj_hbm, proj_v)

    bufs = (b0, b1)
    wsems = (ws0, ws1)
    lanes = lax.iota(jnp.int32, 16)

    def build(ci, p):
        xv = idx_v[pl.ds(ci * C, 16)]
        ws = []
        for i in range(C):
            si = jnp.sum(jnp.where(lanes == i, xv, 0))
            ws.append(jnp.full((16,), si, jnp.int32).astype(jnp.float32))

        def col(j, carry):
            o = j * 16
            p0 = proj_v[pl.ds(o, 16)]
            p1 = proj_v[pl.ds(D_MODEL + o, 16)]
            d = p1 - p0
            for i in range(C):
                bufs[p][pl.ds(i * D_MODEL + o, 16)] = p0 + ws[i] * d
            return carry

        lax.fori_loop(0, D_MODEL // 16, col, 0)

    def start_write(ci, p):
        pltpu.async_copy(
            bufs[p], out_hbm.at[pl.ds((base + ci * C) * D_MODEL, C * D_MODEL)],
            wsems[p])

    def wait_write(ci, p):
        pltpu.make_async_copy(
            bufs[p], out_hbm.at[pl.ds((base + ci * C) * D_MODEL, C * D_MODEL)],
            wsems[p]
        ).wait()

    build(0, 0)

    def body(g, carry):
        ci0 = 2 * g
        start_write(ci0, 0)

        @pl.when(g > 0)
        def _():
            wait_write(ci0 - 1, 1)

        build(ci0 + 1, 1)
        start_write(ci0 + 1, 1)
        wait_write(ci0, 0)

        @pl.when(g < NG - 1)
        def _():
            build(ci0 + 2, 0)

        return carry

    lax.fori_loop(0, NG, body, 0)
    wait_write(NCHUNK - 1, 1)


def kernel(x, emb_table, W, b):
    proj = _compute_proj(emb_table, W, b)
    xf = x.reshape(-1).astype(jnp.int32)
    out = _sc_gather(xf, proj.reshape(-1))
    return out.reshape(x.shape[0], x.shape[1], D_MODEL)


# speedup vs baseline: 6.3202x; 1.0715x over previous
import functools

import jax
import jax.numpy as jnp
from jax import lax
from jax.experimental import pallas as pl
from jax.experimental.pallas import tpu as pltpu
from jax.experimental.pallas import tpu_sc as plsc

D_EMB = 1024
D_MODEL = 2048
N_TOK = 4 * 4096

NC = 2
NS = 16
NW = NC * NS
TW = N_TOK // NW
C = 16
NCHUNK = TW // C
NBUF = 2
NG = NCHUNK // NBUF


def _proj_body(emb_ref, w_ref, b_ref, out_ref):
    out_ref[...] = (
        jnp.dot(emb_ref[...], w_ref[...], preferred_element_type=jnp.float32)
        + b_ref[...][None, :]
    )


def _compute_proj(emb_table, W, b):
    return pl.pallas_call(
        _proj_body,
        out_shape=jax.ShapeDtypeStruct((2, D_MODEL), jnp.float32),
    )(emb_table, W, b)


@functools.partial(
    pl.kernel,
    out_type=jax.ShapeDtypeStruct((N_TOK * D_MODEL,), jnp.float32),
    mesh=plsc.VectorSubcoreMesh(core_axis_name="c", subcore_axis_name="s"),
    compiler_params=pltpu.CompilerParams(needs_layout_passes=False),
    scratch_types=[
        pltpu.VMEM((TW,), jnp.int32),
        pltpu.VMEM((2 * D_MODEL,), jnp.float32),
        pltpu.VMEM((C * D_MODEL,), jnp.float32),
        pltpu.VMEM((C * D_MODEL,), jnp.float32),
        pltpu.SemaphoreType.DMA,
        pltpu.SemaphoreType.DMA,
    ],
)
def _sc_gather(x_hbm, proj_hbm, out_hbm, idx_v, proj_v,
               b0, b1, ws0, ws1):
    cid = lax.axis_index("c")
    sid = lax.axis_index("s")
    wid = sid * NC + cid
    base = wid * TW
    pltpu.sync_copy(x_hbm.at[pl.ds(base, TW)], idx_v)
    pltpu.sync_copy(proj_hbm, proj_v)

    bufs = (b0, b1)
    wsems = (ws0, ws1)
    lanes = lax.iota(jnp.int32, 16)

    def build(ci, p):
        xv = idx_v[pl.ds(ci * C, 16)]
        ws = []
        for i in range(C):
            si = jnp.sum(jnp.where(lanes == i, xv, 0))
            ws.append(jnp.full((16,), si, jnp.int32).astype(jnp.float32))

        def col(j, carry):
            o = j * 16
            p0 = proj_v[pl.ds(o, 16)]
            p1 = proj_v[pl.ds(D_MODEL + o, 16)]
            d = p1 - p0
            for i in range(C):
                bufs[p][pl.ds(i * D_MODEL + o, 16)] = p0 + ws[i] * d
            return carry

        lax.fori_loop(0, D_MODEL // 16, col, 0, unroll=4)

    def start_write(ci, p):
        pltpu.async_copy(
            bufs[p], out_hbm.at[pl.ds((base + ci * C) * D_MODEL, C * D_MODEL)],
            wsems[p])

    def wait_write(ci, p):
        pltpu.make_async_copy(
            bufs[p], out_hbm.at[pl.ds((base + ci * C) * D_MODEL, C * D_MODEL)],
            wsems[p]
        ).wait()

    build(0, 0)

    def body(g, carry):
        ci0 = 2 * g
        start_write(ci0, 0)

        @pl.when(g > 0)
        def _():
            wait_write(ci0 - 1, 1)

        build(ci0 + 1, 1)
        start_write(ci0 + 1, 1)
        wait_write(ci0, 0)

        @pl.when(g < NG - 1)
        def _():
            build(ci0 + 2, 0)

        return carry

    lax.fori_loop(0, NG, body, 0)
    wait_write(NCHUNK - 1, 1)


def kernel(x, emb_table, W, b):
    proj = _compute_proj(emb_table, W, b)
    xf = x.reshape(-1).astype(jnp.int32)
    out = _sc_gather(xf, proj.reshape(-1))
    return out.reshape(x.shape[0], x.shape[1], D_MODEL)


# speedup vs baseline: 6.6735x; 1.0559x over previous
import functools

import jax
import jax.numpy as jnp
from jax import lax
from jax.experimental import pallas as pl
from jax.experimental.pallas import tpu as pltpu
from jax.experimental.pallas import tpu_sc as plsc

D_EMB = 1024
D_MODEL = 2048
N_TOK = 4 * 4096

NC = 2
NS = 16
NW = NC * NS
TW = N_TOK // NW
C = 16
NCHUNK = TW // C
NBUF = 2
NG = NCHUNK // NBUF


def _proj_body(emb_ref, w_ref, b_ref, out_ref):
    out_ref[...] = (
        jnp.dot(emb_ref[...], w_ref[...], preferred_element_type=jnp.float32)
        + b_ref[...][None, :]
    )


def _compute_proj(emb_table, W, b):
    return pl.pallas_call(
        _proj_body,
        out_shape=jax.ShapeDtypeStruct((2, D_MODEL), jnp.float32),
    )(emb_table, W, b)


@functools.partial(
    pl.kernel,
    out_type=jax.ShapeDtypeStruct((N_TOK * D_MODEL,), jnp.float32),
    mesh=plsc.VectorSubcoreMesh(core_axis_name="c", subcore_axis_name="s"),
    compiler_params=pltpu.CompilerParams(needs_layout_passes=False),
    scratch_types=[
        pltpu.VMEM((TW,), jnp.int32),
        pltpu.VMEM((2 * D_MODEL,), jnp.float32),
        pltpu.VMEM((C * D_MODEL,), jnp.float32),
        pltpu.SemaphoreType.DMA,
    ],
)
def _sc_gather(x_hbm, proj_hbm, out_hbm, idx_v, proj_v, dbuf, sem):
    cid = lax.axis_index("c")
    sid = lax.axis_index("s")
    wid = sid * NC + cid
    base = wid * TW
    pltpu.sync_copy(x_hbm.at[pl.ds(base, TW)], idx_v)
    pltpu.sync_copy(proj_hbm, proj_v)

    GRP = 16
    NGRP = TW // GRP
    LAG = 8

    def drain_one():
        pltpu.make_async_copy(
            out_hbm.at[pl.ds(base * D_MODEL, GRP * D_MODEL)], dbuf, sem
        ).wait()

    def tok_grp(g, carry):
        v = idx_v[pl.ds(g * GRP, GRP)]
        for i in range(GRP):
            t = g * GRP + i
            src_off = v[i] * D_MODEL
            pltpu.async_copy(
                proj_v.at[pl.ds(src_off, D_MODEL)],
                out_hbm.at[pl.ds((base + t) * D_MODEL, D_MODEL)],
                sem,
            )

        @pl.when(g >= LAG)
        def _():
            drain_one()

        return carry

    lax.fori_loop(0, NGRP, tok_grp, 0)

    def tail(g, carry):
        drain_one()
        return carry

    lax.fori_loop(0, LAG, tail, 0)


def kernel(x, emb_table, W, b):
    proj = _compute_proj(emb_table, W, b)
    xf = x.reshape(-1).astype(jnp.int32)
    out = _sc_gather(xf, proj.reshape(-1))
    return out.reshape(x.shape[0], x.shape[1], D_MODEL)
